# initial kernel scaffold (unmeasured)
import jax
import jax.numpy as jnp
from jax import lax
from jax.experimental import pallas as pl
from jax.experimental.pallas import tpu as pltpu

N_DEV = 4
M_BLK = 1024
K_BLK = 1024
N_TOT = 8192
N_CHUNK = 2048


def _body(x_ref, w_ref, out_ref, gath_ref, send_sems, recv_sems):
    my = lax.axis_index("i")

    barrier = pltpu.get_barrier_semaphore()
    for d in range(1, N_DEV):
        peer = lax.rem(my + d, N_DEV)
        pl.semaphore_signal(
            barrier, inc=1,
            device_id=(peer,), device_id_type=pl.DeviceIdType.MESH,
        )
    pl.semaphore_wait(barrier, N_DEV - 1)

    sends = []
    for d in range(1, N_DEV):
        t = lax.rem(my + d, N_DEV)
        rdma = pltpu.make_async_remote_copy(
            src_ref=x_ref.at[t],
            dst_ref=gath_ref.at[my],
            send_sem=send_sems.at[d - 1],
            recv_sem=recv_sems.at[d - 1],
            device_id=(t,),
            device_id_type=pl.DeviceIdType.MESH,
        )
        rdma.start()
        sends.append(rdma)

    def accum(k_idx, lhs, first):
        for c in range(N_TOT // N_CHUNK):
            nsl = pl.ds(c * N_CHUNK, N_CHUNK)
            part = jnp.dot(
                lhs,
                w_ref[pl.ds(k_idx * K_BLK, K_BLK), nsl],
                preferred_element_type=jnp.float32,
            )
            if first:
                out_ref[:, nsl] = part
            else:
                out_ref[:, nsl] = out_ref[:, nsl] + part

    accum(my, x_ref[my], first=True)

    for d in (1, 3, 2):
        s = lax.rem(my + (N_DEV - d), N_DEV)
        recv = pltpu.make_async_remote_copy(
            src_ref=x_ref.at[s],
            dst_ref=gath_ref.at[s],
            send_sem=send_sems.at[d - 1],
            recv_sem=recv_sems.at[d - 1],
            device_id=(s,),
            device_id_type=pl.DeviceIdType.MESH,
        )
        recv.wait_recv()
        accum(s, gath_ref[s], first=False)

    for rdma in sends:
        rdma.wait_send()


def kernel(x, w_mat):
    x16 = x.astype(jnp.bfloat16).reshape(N_DEV, M_BLK, K_BLK)
    w16 = w_mat.astype(jnp.bfloat16)
    return pl.pallas_call(
        _body,
        out_shape=jax.ShapeDtypeStruct((M_BLK, N_TOT), jnp.float32),
        in_specs=[
            pl.BlockSpec(memory_space=pltpu.VMEM),
            pl.BlockSpec(memory_space=pltpu.VMEM),
        ],
        out_specs=pl.BlockSpec(memory_space=pltpu.VMEM),
        scratch_shapes=[
            pltpu.VMEM((N_DEV, M_BLK, K_BLK), jnp.bfloat16),
            pltpu.SemaphoreType.DMA((N_DEV - 1,)),
            pltpu.SemaphoreType.DMA((N_DEV - 1,)),
        ],
        compiler_params=pltpu.CompilerParams(collective_id=0),
    )(x16, w16)


# baseline (device time: 145794 ns/iter reference)
import jax
import jax.numpy as jnp
from jax import lax
from jax.experimental import pallas as pl
from jax.experimental.pallas import tpu as pltpu

N_DEV = 4
M_BLK = 1024
K_BLK = 1024
N_TOT = 8192
N_CHUNK = 512
N_CHUNKS = N_TOT // N_CHUNK
N_TILES = N_DEV * N_CHUNKS


def _body(x_ref, w_ref, out_ref, gath_ref, w_buf, send_sems, recv_sems, w_sems):
    my = lax.axis_index("i")

    barrier = pltpu.get_barrier_semaphore()
    for d in range(1, N_DEV):
        peer = lax.rem(my + d, N_DEV)
        pl.semaphore_signal(
            barrier, inc=1,
            device_id=(peer,), device_id_type=pl.DeviceIdType.MESH,
        )
    pl.semaphore_wait(barrier, N_DEV - 1)

    sends = []
    for d in range(1, N_DEV):
        t = lax.rem(my + d, N_DEV)
        rdma = pltpu.make_async_remote_copy(
            src_ref=x_ref.at[t],
            dst_ref=gath_ref.at[d - 1],
            send_sem=send_sems.at[d - 1],
            recv_sem=recv_sems.at[d - 1],
            device_id=(t,),
            device_id_type=pl.DeviceIdType.MESH,
        )
        rdma.start()
        sends.append(rdma)

    d_for_phase = [None, 1, 3, 2]
    k_order = [my] + [lax.rem(my + (N_DEV - d), N_DEV) for d in d_for_phase[1:]]

    def w_dma(t):
        p, c = divmod(t, N_CHUNKS)
        return pltpu.make_async_copy(
            w_ref.at[pl.ds(k_order[p] * K_BLK, K_BLK),
                     pl.ds(c * N_CHUNK, N_CHUNK)],
            w_buf.at[t % 2],
            w_sems.at[t % 2],
        )

    w_dma(0).start()
    w_dma(1).start()

    lhs = x_ref[my]
    for t in range(N_TILES):
        p, c = divmod(t, N_CHUNKS)
        if c == 0 and p > 0:
            d = d_for_phase[p]
            recv = pltpu.make_async_remote_copy(
                src_ref=x_ref.at[k_order[p]],
                dst_ref=gath_ref.at[d - 1],
                send_sem=send_sems.at[d - 1],
                recv_sem=recv_sems.at[d - 1],
                device_id=(k_order[p],),
                device_id_type=pl.DeviceIdType.MESH,
            )
            recv.wait_recv()
            lhs = gath_ref[d - 1]
        w_dma(t).wait()
        wt = w_buf[t % 2].astype(jnp.bfloat16)
        nsl = pl.ds(c * N_CHUNK, N_CHUNK)
        part = jnp.dot(lhs, wt, preferred_element_type=jnp.float32)
        if p == 0:
            out_ref[:, nsl] = part
        else:
            out_ref[:, nsl] = out_ref[:, nsl] + part
        if t + 2 < N_TILES:
            w_dma(t + 2).start()

    for rdma in sends:
        rdma.wait_send()


def kernel(x, w_mat):
    x16 = x.astype(jnp.bfloat16).reshape(N_DEV, M_BLK, K_BLK)
    return pl.pallas_call(
        _body,
        out_shape=jax.ShapeDtypeStruct((M_BLK, N_TOT), jnp.float32),
        in_specs=[
            pl.BlockSpec(memory_space=pltpu.VMEM),
            pl.BlockSpec(memory_space=pltpu.HBM),
        ],
        out_specs=pl.BlockSpec(memory_space=pltpu.VMEM),
        scratch_shapes=[
            pltpu.VMEM((N_DEV - 1, M_BLK, K_BLK), jnp.bfloat16),
            pltpu.VMEM((2, K_BLK, N_CHUNK), jnp.float32),
            pltpu.SemaphoreType.DMA((N_DEV - 1,)),
            pltpu.SemaphoreType.DMA((N_DEV - 1,)),
            pltpu.SemaphoreType.DMA((2,)),
        ],
        compiler_params=pltpu.CompilerParams(
            collective_id=0,
            vmem_limit_bytes=60 * 1024 * 1024,
        ),
    )(x16, w_mat)
